# baseline (speedup 1.0000x reference)
import functools

import jax
import jax.numpy as jnp
from jax import lax
from jax.experimental import pallas as pl
from jax.experimental.pallas import tpu as pltpu
from jax.experimental.pallas import tpu_sc as plsc
from jax.experimental.layout import Format, Layout, with_layout_constraint

B = 16384
D = 64
H = 2 * D
V = 100000
NC = 2
NS = 16
NW = NC * NS
BPW = B // NW
L = 16
TB = 1024
NB = B // TB
AVG_R = 3.5
LAM = 0.001


def _sc_gather(user0, item_i0, ratings, eu8, ei8, user_bias, item_bias):
    mesh = plsc.VectorSubcoreMesh(core_axis_name="c", subcore_axis_name="s")

    @functools.partial(
        pl.kernel,
        mesh=mesh,
        compiler_params=pltpu.CompilerParams(use_tc_tiling_on_sc=False),
        out_type=(
            jax.ShapeDtypeStruct((B, 128), jnp.float32),
            jax.ShapeDtypeStruct((B,), jnp.float32),
        ),
        scratch_types=(
            pltpu.VMEM((BPW,), jnp.int32),
            pltpu.VMEM((BPW,), jnp.int32),
            pltpu.VMEM((BPW, D), jnp.float32),
            pltpu.VMEM((BPW, D), jnp.float32),
            pltpu.VMEM((BPW,), jnp.float32),
            pltpu.VMEM((BPW,), jnp.float32),
            pltpu.VMEM((BPW,), jnp.float32),
            pltpu.VMEM((BPW,), jnp.float32),
            pltpu.SemaphoreType.DMA,
            pltpu.SemaphoreType.DMA,
            pltpu.SemaphoreType.DMA,
            pltpu.SemaphoreType.DMA,
            pltpu.SemaphoreType.DMA,
        ),
    )
    def gather_kernel(u0_hbm, i0_hbm, rat_hbm, eu_hbm, ei_hbm, ubt_hbm,
                      ibt_hbm,
                      rows_out, c_out,
                      uidx_v, iidx_v, urows_v, irows_v, ubv, ibv, ratv, cv,
                      sem_u, sem_i, sem_ub, sem_ib, sem_r):
        wid = lax.axis_index("s") * NC + lax.axis_index("c")
        base = pl.multiple_of(wid * BPW, BPW)
        pltpu.sync_copy(u0_hbm.at[pl.ds(base, BPW)], uidx_v)
        pltpu.sync_copy(i0_hbm.at[pl.ds(base, BPW)], iidx_v)
        cu = pltpu.async_copy(eu_hbm.at[uidx_v], urows_v, sem_u)
        ci = pltpu.async_copy(ei_hbm.at[iidx_v], irows_v, sem_i)
        cub = pltpu.async_copy(ubt_hbm.at[uidx_v], ubv, sem_ub)
        cib = pltpu.async_copy(ibt_hbm.at[iidx_v], ibv, sem_ib)
        crat = pltpu.async_copy(rat_hbm.at[pl.ds(base, BPW)], ratv, sem_r)
        cub.wait()
        cib.wait()
        crat.wait()

        def c_chunk(k, carry):
            s = pl.multiple_of(k * L, L)
            cv[pl.ds(s, L)] = (ubv[pl.ds(s, L)] + ibv[pl.ds(s, L)]
                               + AVG_R - ratv[pl.ds(s, L)])
            return carry

        lax.fori_loop(0, BPW // L, c_chunk, 0, unroll=4)
        pltpu.sync_copy(cv, c_out.at[pl.ds(base, BPW)])
        cu.wait()
        pltpu.sync_copy(urows_v,
                        rows_out.at[pl.ds(base, BPW), pl.ds(0, D)])
        ci.wait()
        pltpu.sync_copy(irows_v,
                        rows_out.at[pl.ds(base, BPW), pl.ds(D, D)])

    return gather_kernel(user0, item_i0, ratings, eu8, ei8,
                         user_bias, item_bias)


def _tc_body(x_ref, c_ref, W1_ref, b1_ref, W2_ref, b2_ref, part_ref):
    W1 = W1_ref[...]
    b1 = b1_ref[...]
    W2 = W2_ref[...]
    b2 = b2_ref[...]

    def mlp(x):
        h = jnp.dot(x, W1, preferred_element_type=jnp.float32) + b1
        h = jnp.where(h >= 0, h, 0.1 * h)
        return jnp.dot(h, W2, preferred_element_type=jnp.float32) + b2

    x = x_ref[...]
    fu = mlp(x[:, :D])
    fi = mlp(x[:, D:])
    dots = jnp.sum(fu * fi, axis=1)
    err = dots + c_ref[...]
    i = pl.program_id(0)
    part_ref[i, 0] = jnp.sum(err * err)
    part_ref[i, 1] = jnp.sum(fu * fu)
    part_ref[i, 2] = jnp.sum(fi * fi)


def _tc_loss(rows, c, W1, b1, W2, b2):
    return pl.pallas_call(
        _tc_body,
        grid=(NB,),
        in_specs=[
            pl.BlockSpec((TB, 128), lambda i: (i, 0)),
            pl.BlockSpec((TB,), lambda i: (i,)),
            pl.BlockSpec((D, H), lambda i: (0, 0)),
            pl.BlockSpec((1, H), lambda i: (0, 0)),
            pl.BlockSpec((H, D), lambda i: (0, 0)),
            pl.BlockSpec((1, D), lambda i: (0, 0)),
        ],
        out_specs=pl.BlockSpec(memory_space=pltpu.SMEM),
        out_shape=jax.ShapeDtypeStruct((NB, 3), jnp.float32),
    )(rows, c, W1, b1, W2, b2)


def kernel(user0, item_i0, ratings, embed_user, embed_item,
           W1, b1, W2, b2, user_bias, item_bias):
    u0 = user0.astype(jnp.int32)
    i0 = item_i0.astype(jnp.int32)
    t8 = Layout(major_to_minor=(0, 1), tiling=((8,),))
    eu8 = with_layout_constraint(embed_user, t8)
    ei8 = with_layout_constraint(embed_item, t8)
    rows, c = _sc_gather(u0, i0, ratings.astype(jnp.float32), eu8, ei8,
                         user_bias[:, 0], item_bias[:, 0])
    parts = _tc_loss(rows, c, W1, b1.reshape(1, H), W2, b2.reshape(1, D))
    sums = jnp.sum(parts, axis=0)
    loss2 = sums[0] / B
    l2 = LAM * (sums[1] / (B * D)) + LAM * (sums[2] / (B * D))
    loss = loss2 + l2
    z = jnp.float32(0.0)
    return (loss, loss2, z, z, z, z)


# speedup vs baseline: 1.1150x; 1.1150x over previous
import functools

import jax
import jax.numpy as jnp
from jax import lax
from jax.experimental import pallas as pl
from jax.experimental.pallas import tpu as pltpu
from jax.experimental.pallas import tpu_sc as plsc
from jax.experimental.layout import Format, Layout, with_layout_constraint

B = 16384
D = 64
H = 2 * D
V = 100000
NC = 2
NS = 16
NW = NC * NS
BPW = B // NW
L = 16
TB = 4096
NB = B // TB
AVG_R = 3.5
LAM = 0.001


def _sc_gather(user0, item_i0, ratings, eu8, ei8, user_bias, item_bias):
    mesh = plsc.VectorSubcoreMesh(core_axis_name="c", subcore_axis_name="s")

    @functools.partial(
        pl.kernel,
        mesh=mesh,
        compiler_params=pltpu.CompilerParams(use_tc_tiling_on_sc=False),
        out_type=(
            jax.ShapeDtypeStruct((B, 128), jnp.float32),
            jax.ShapeDtypeStruct((B,), jnp.float32),
        ),
        scratch_types=(
            pltpu.VMEM((BPW,), jnp.int32),
            pltpu.VMEM((BPW,), jnp.int32),
            pltpu.VMEM((BPW, D), jnp.float32),
            pltpu.VMEM((BPW, D), jnp.float32),
            pltpu.VMEM((BPW,), jnp.float32),
            pltpu.VMEM((BPW,), jnp.float32),
            pltpu.VMEM((BPW,), jnp.float32),
            pltpu.VMEM((BPW,), jnp.float32),
            pltpu.SemaphoreType.DMA,
            pltpu.SemaphoreType.DMA,
            pltpu.SemaphoreType.DMA,
            pltpu.SemaphoreType.DMA,
            pltpu.SemaphoreType.DMA,
        ),
    )
    def gather_kernel(u0_hbm, i0_hbm, rat_hbm, eu_hbm, ei_hbm, ubt_hbm,
                      ibt_hbm,
                      rows_out, c_out,
                      uidx_v, iidx_v, urows_v, irows_v, ubv, ibv, ratv, cv,
                      sem_u, sem_i, sem_ub, sem_ib, sem_r):
        wid = lax.axis_index("s") * NC + lax.axis_index("c")
        base = pl.multiple_of(wid * BPW, BPW)
        pltpu.sync_copy(u0_hbm.at[pl.ds(base, BPW)], uidx_v)
        pltpu.sync_copy(i0_hbm.at[pl.ds(base, BPW)], iidx_v)
        cu = pltpu.async_copy(eu_hbm.at[uidx_v], urows_v, sem_u)
        ci = pltpu.async_copy(ei_hbm.at[iidx_v], irows_v, sem_i)
        cub = pltpu.async_copy(ubt_hbm.at[uidx_v], ubv, sem_ub)
        cib = pltpu.async_copy(ibt_hbm.at[iidx_v], ibv, sem_ib)
        crat = pltpu.async_copy(rat_hbm.at[pl.ds(base, BPW)], ratv, sem_r)
        cub.wait()
        cib.wait()
        crat.wait()

        def c_chunk(k, carry):
            s = pl.multiple_of(k * L, L)
            cv[pl.ds(s, L)] = (ubv[pl.ds(s, L)] + ibv[pl.ds(s, L)]
                               + AVG_R - ratv[pl.ds(s, L)])
            return carry

        lax.fori_loop(0, BPW // L, c_chunk, 0, unroll=4)
        pltpu.sync_copy(cv, c_out.at[pl.ds(base, BPW)])
        cu.wait()
        pltpu.sync_copy(urows_v,
                        rows_out.at[pl.ds(base, BPW), pl.ds(0, D)])
        ci.wait()
        pltpu.sync_copy(irows_v,
                        rows_out.at[pl.ds(base, BPW), pl.ds(D, D)])

    return gather_kernel(user0, item_i0, ratings, eu8, ei8,
                         user_bias, item_bias)


def _tc_body(x_ref, c_ref, W1_ref, b1_ref, W2_ref, b2_ref, part_ref):
    W1 = W1_ref[...]
    b1 = b1_ref[...]
    W2 = W2_ref[...]
    b2 = b2_ref[...]

    def mlp(x):
        h = jnp.dot(x, W1, preferred_element_type=jnp.float32) + b1
        h = jnp.where(h >= 0, h, 0.1 * h)
        return jnp.dot(h, W2, preferred_element_type=jnp.float32) + b2

    x = x_ref[...]
    fu = mlp(x[:, :D])
    fi = mlp(x[:, D:])
    dots = jnp.sum(fu * fi, axis=1)
    err = dots + c_ref[...]
    i = pl.program_id(0)
    part_ref[i, 0] = jnp.sum(err * err)
    part_ref[i, 1] = jnp.sum(fu * fu)
    part_ref[i, 2] = jnp.sum(fi * fi)


def _tc_loss(rows, c, W1, b1, W2, b2):
    return pl.pallas_call(
        _tc_body,
        grid=(NB,),
        in_specs=[
            pl.BlockSpec((TB, 128), lambda i: (i, 0)),
            pl.BlockSpec((TB,), lambda i: (i,)),
            pl.BlockSpec((D, H), lambda i: (0, 0)),
            pl.BlockSpec((1, H), lambda i: (0, 0)),
            pl.BlockSpec((H, D), lambda i: (0, 0)),
            pl.BlockSpec((1, D), lambda i: (0, 0)),
        ],
        out_specs=pl.BlockSpec(memory_space=pltpu.SMEM),
        out_shape=jax.ShapeDtypeStruct((NB, 3), jnp.float32),
    )(rows, c, W1, b1, W2, b2)


def kernel(user0, item_i0, ratings, embed_user, embed_item,
           W1, b1, W2, b2, user_bias, item_bias):
    u0 = user0.astype(jnp.int32)
    i0 = item_i0.astype(jnp.int32)
    t8 = Layout(major_to_minor=(0, 1), tiling=((8,),))
    eu8 = with_layout_constraint(embed_user, t8)
    ei8 = with_layout_constraint(embed_item, t8)
    rows, c = _sc_gather(u0, i0, ratings.astype(jnp.float32), eu8, ei8,
                         user_bias[:, 0], item_bias[:, 0])
    parts = _tc_loss(rows, c, W1, b1.reshape(1, H), W2, b2.reshape(1, D))
    sums = jnp.sum(parts, axis=0)
    loss2 = sums[0] / B
    l2 = LAM * (sums[1] / (B * D)) + LAM * (sums[2] / (B * D))
    loss = loss2 + l2
    z = jnp.float32(0.0)
    return (loss, loss2, z, z, z, z)
